# baseline (speedup 1.0000x reference)
import jax
import jax.numpy as jnp
from jax.experimental import pallas as pl
from jax.experimental.pallas import tpu as pltpu

_EPS = 1e-5


def _shift_w(v, d):
    if d == 0:
        return v
    z = jnp.zeros(v.shape[:-1] + (1,), v.dtype)
    if d == 1:
        return jnp.concatenate([v[..., 1:], z], axis=-1)
    return jnp.concatenate([z, v[..., :-1]], axis=-1)


def _down_body(x_hbm, w1_ref, b1_ref, w2_ref, b2_ref, sel_ref, o_ref,
               xin, psh, a1, a2):
    n = pl.program_id(0)
    r = pl.program_id(1)
    num_r = pl.num_programs(1)
    cout = o_ref.shape[1]
    cin = xin.shape[0]
    nrows = xin.shape[1]
    w2x = xin.shape[2]
    w = w2x // 2
    wp = w // 2
    th = o_ref.shape[2] // wp

    @pl.when(r == 0)
    def _():
        xin[:, 0:8, :] = jnp.zeros((cin, 8, w2x), xin.dtype)
        pltpu.sync_copy(x_hbm.at[n, :, pl.ds(0, th + 8), :],
                        xin.at[:, pl.ds(8, th + 8), :])

    @pl.when(jnp.logical_and(r > 0, r < num_r - 1))
    def _():
        pltpu.sync_copy(x_hbm.at[n, :, pl.ds(r * th - 8, th + 16), :],
                        xin.at[:, pl.ds(0, th + 16), :])

    @pl.when(jnp.logical_and(r == num_r - 1, num_r > 1))
    def _():
        pltpu.sync_copy(x_hbm.at[n, :, pl.ds(r * th - 8, th + 8), :],
                        xin.at[:, pl.ds(0, th + 8), :])
        xin[:, th + 8:th + 16, :] = jnp.zeros((cin, 8, w2x), xin.dtype)

    v = xin[...]
    hm = jnp.maximum(v[:, :, 0:w], v[:, :, w:w2x])
    wm = jnp.maximum(hm, _shift_w(hm, 1)).astype(jnp.bfloat16)
    pr = jnp.dot(wm.reshape(cin * nrows, w), sel_ref[...],
                 preferred_element_type=jnp.float32).astype(jnp.bfloat16)
    pr = pr.reshape(cin, nrows, wp)
    psh[1] = pr
    psh[0] = _shift_w(pr, -1)
    psh[2] = _shift_w(pr, 1)

    for j in range(th + 2):
        for ky in range(3):
            for kx in range(3):
                t = ky * 3 + kx
                a1[t * cin:(t + 1) * cin, pl.ds(j * wp, wp)] = \
                    psh[kx, :, j + ky + 6, :]

    y1 = jnp.dot(w1_ref[...], a1[...], preferred_element_type=jnp.float32)
    y1 = jnp.maximum(y1 + b1_ref[:, 0:1], 0.0).astype(jnp.bfloat16)

    pos = jax.lax.broadcasted_iota(jnp.int32, (1, (th + 2) * wp), 1)
    zero = jnp.zeros((), jnp.bfloat16)
    posw = pos % wp
    y1_0 = jnp.where(posw == 0, zero, _shift_w(y1, -1))
    y1_2 = jnp.where(posw == wp - 1, zero, _shift_w(y1, 1))

    for kx, yv in ((0, y1_0), (1, y1), (2, y1_2)):
        for ky in range(3):
            t = ky * 3 + kx
            a2[t * cout:(t + 1) * cout, :] = yv[:, ky * wp:(ky + th) * wp]

    @pl.when(r == 0)
    def _():
        a2[0:3 * cout, 0:wp] = jnp.zeros((3 * cout, wp), a2.dtype)

    @pl.when(r == num_r - 1)
    def _():
        a2[6 * cout:9 * cout, (th - 1) * wp:th * wp] = jnp.zeros(
            (3 * cout, wp), a2.dtype)

    y2 = jnp.dot(w2_ref[...], a2[...], preferred_element_type=jnp.float32)
    y2 = jnp.maximum(y2 + b2_ref[:, 0:1], 0.0)
    o_ref[0] = y2.astype(jnp.bfloat16).astype(jnp.float32)


def _fold_bn(w, b, gamma, beta, mean, var, wp):
    kh, kw, cin, cout = w.shape
    scale = gamma / jnp.sqrt(var + _EPS)
    w_eff = w * scale[None, None, None, :]
    b_eff = (b - mean) * scale + beta
    wm = jnp.transpose(w_eff.reshape(kh * kw * cin, cout)).astype(jnp.bfloat16)
    bb = jnp.broadcast_to(b_eff[:, None], (cout, wp))
    return wm, bb


def kernel(x, w1, b1, gamma1, beta1, mean1, var1,
           w2, b2, gamma2, beta2, mean2, var2):
    N, Cin, H, W = x.shape
    Cout = w1.shape[-1]
    Hp, Wp = H // 2, W // 2

    th = min(32, Hp)
    while Hp % th:
        th -= 1
    R = Hp // th

    w1m, b1b = _fold_bn(w1, b1, gamma1, beta1, mean1, var1, Wp)
    w2m, b2b = _fold_bn(w2, b2, gamma2, beta2, mean2, var2, Wp)
    sel = (jnp.arange(W)[:, None] == 2 * jnp.arange(Wp)[None, :]
           ).astype(jnp.bfloat16)

    grid_spec = pltpu.PrefetchScalarGridSpec(
        num_scalar_prefetch=0,
        grid=(N, R),
        in_specs=[
            pl.BlockSpec(memory_space=pl.ANY),
            pl.BlockSpec((Cout, 9 * Cin), lambda n, r: (0, 0)),
            pl.BlockSpec((Cout, Wp), lambda n, r: (0, 0)),
            pl.BlockSpec((Cout, 9 * Cout), lambda n, r: (0, 0)),
            pl.BlockSpec((Cout, Wp), lambda n, r: (0, 0)),
            pl.BlockSpec((W, Wp), lambda n, r: (0, 0)),
        ],
        out_specs=pl.BlockSpec((1, Cout, th * Wp), lambda n, r: (n, 0, r)),
        scratch_shapes=[
            pltpu.VMEM((Cin, th + 16, 2 * W), jnp.float32),
            pltpu.VMEM((3, Cin, th + 16, Wp), jnp.bfloat16),
            pltpu.VMEM((9 * Cin, (th + 2) * Wp), jnp.bfloat16),
            pltpu.VMEM((9 * Cout, th * Wp), jnp.bfloat16),
        ],
    )

    y = pl.pallas_call(
        _down_body,
        grid_spec=grid_spec,
        out_shape=jax.ShapeDtypeStruct((N, Cout, Hp * Wp), jnp.float32),
        compiler_params=pltpu.CompilerParams(
            dimension_semantics=("parallel", "parallel"),
            vmem_limit_bytes=56 * 2**20),
    )(x.reshape(N, Cin, Hp, 2 * W), w1m, b1b, w2m, b2b, sel)
    return y.reshape(N, Cout, Hp, Wp)


# speedup vs baseline: 1.1629x; 1.1629x over previous
import jax
import jax.numpy as jnp
from jax.experimental import pallas as pl
from jax.experimental.pallas import tpu as pltpu

_EPS = 1e-5


def _shift_w(v, d):
    if d == 0:
        return v
    z = jnp.zeros(v.shape[:-1] + (1,), v.dtype)
    if d == 1:
        return jnp.concatenate([v[..., 1:], z], axis=-1)
    return jnp.concatenate([z, v[..., :-1]], axis=-1)


def _down_body(x_hbm, w1_ref, b1_ref, w2_ref, b2_ref, sel_ref, o_ref,
               xin, psh, a1, a2):
    n = pl.program_id(0)
    r = pl.program_id(1)
    num_r = pl.num_programs(1)
    cout = o_ref.shape[1]
    cin = xin.shape[0]
    nrows = xin.shape[1]
    w2x = xin.shape[2]
    w = w2x // 2
    wp = w // 2
    th = o_ref.shape[2] // wp

    @pl.when(r == 0)
    def _():
        xin[:, 0:8, :] = jnp.zeros((cin, 8, w2x), xin.dtype)
        pltpu.sync_copy(x_hbm.at[n, :, pl.ds(0, th + 8), :],
                        xin.at[:, pl.ds(8, th + 8), :])

    @pl.when(jnp.logical_and(r > 0, r < num_r - 1))
    def _():
        pltpu.sync_copy(x_hbm.at[n, :, pl.ds(r * th - 8, th + 16), :],
                        xin.at[:, pl.ds(0, th + 16), :])

    @pl.when(jnp.logical_and(r == num_r - 1, num_r > 1))
    def _():
        pltpu.sync_copy(x_hbm.at[n, :, pl.ds(r * th - 8, th + 8), :],
                        xin.at[:, pl.ds(0, th + 8), :])
        xin[:, th + 8:th + 16, :] = jnp.zeros((cin, 8, w2x), xin.dtype)

    v = xin[...]
    hm = jnp.maximum(v[:, :, 0:w], v[:, :, w:w2x])
    psh[...] = jnp.maximum(hm, _shift_w(hm, 1)).astype(jnp.bfloat16)

    for i in range(th + 4):
        m = psh[:, i + 6, :]
        p1 = jnp.dot(m, sel_ref[...],
                     preferred_element_type=jnp.float32).astype(jnp.bfloat16)
        pv = (_shift_w(p1, -1), p1, _shift_w(p1, 1))
        for ky in range(3):
            j = i - ky
            if 0 <= j < th + 2:
                for kx in range(3):
                    t = ky * 3 + kx
                    a1[t * cin:(t + 1) * cin, pl.ds(j * wp, wp)] = pv[kx]

    y1 = jnp.dot(w1_ref[...], a1[...], preferred_element_type=jnp.float32)
    y1 = jnp.maximum(y1 + b1_ref[:, 0:1], 0.0).astype(jnp.bfloat16)

    pos = jax.lax.broadcasted_iota(jnp.int32, (1, (th + 2) * wp), 1)
    zero = jnp.zeros((), jnp.bfloat16)
    posw = pos % wp
    y1_0 = jnp.where(posw == 0, zero, _shift_w(y1, -1))
    y1_2 = jnp.where(posw == wp - 1, zero, _shift_w(y1, 1))

    for kx, yv in ((0, y1_0), (1, y1), (2, y1_2)):
        for ky in range(3):
            t = ky * 3 + kx
            a2[t * cout:(t + 1) * cout, :] = yv[:, ky * wp:(ky + th) * wp]

    @pl.when(r == 0)
    def _():
        a2[0:3 * cout, 0:wp] = jnp.zeros((3 * cout, wp), a2.dtype)

    @pl.when(r == num_r - 1)
    def _():
        a2[6 * cout:9 * cout, (th - 1) * wp:th * wp] = jnp.zeros(
            (3 * cout, wp), a2.dtype)

    y2 = jnp.dot(w2_ref[...], a2[...], preferred_element_type=jnp.float32)
    y2 = jnp.maximum(y2 + b2_ref[:, 0:1], 0.0)
    o_ref[0] = y2.astype(jnp.bfloat16).astype(jnp.float32)


def _fold_bn(w, b, gamma, beta, mean, var, wp):
    kh, kw, cin, cout = w.shape
    scale = gamma / jnp.sqrt(var + _EPS)
    w_eff = w * scale[None, None, None, :]
    b_eff = (b - mean) * scale + beta
    wm = jnp.transpose(w_eff.reshape(kh * kw * cin, cout)).astype(jnp.bfloat16)
    bb = jnp.broadcast_to(b_eff[:, None], (cout, wp))
    return wm, bb


def kernel(x, w1, b1, gamma1, beta1, mean1, var1,
           w2, b2, gamma2, beta2, mean2, var2):
    N, Cin, H, W = x.shape
    Cout = w1.shape[-1]
    Hp, Wp = H // 2, W // 2

    th = min(32, Hp)
    while Hp % th:
        th -= 1
    R = Hp // th

    w1m, b1b = _fold_bn(w1, b1, gamma1, beta1, mean1, var1, Wp)
    w2m, b2b = _fold_bn(w2, b2, gamma2, beta2, mean2, var2, Wp)
    sel = (jnp.arange(W)[:, None] == 2 * jnp.arange(Wp)[None, :]
           ).astype(jnp.bfloat16)

    grid_spec = pltpu.PrefetchScalarGridSpec(
        num_scalar_prefetch=0,
        grid=(N, R),
        in_specs=[
            pl.BlockSpec(memory_space=pl.ANY),
            pl.BlockSpec((Cout, 9 * Cin), lambda n, r: (0, 0)),
            pl.BlockSpec((Cout, Wp), lambda n, r: (0, 0)),
            pl.BlockSpec((Cout, 9 * Cout), lambda n, r: (0, 0)),
            pl.BlockSpec((Cout, Wp), lambda n, r: (0, 0)),
            pl.BlockSpec((W, Wp), lambda n, r: (0, 0)),
        ],
        out_specs=pl.BlockSpec((1, Cout, th * Wp), lambda n, r: (n, 0, r)),
        scratch_shapes=[
            pltpu.VMEM((Cin, th + 16, 2 * W), jnp.float32),
            pltpu.VMEM((Cin, th + 16, W), jnp.bfloat16),
            pltpu.VMEM((9 * Cin, (th + 2) * Wp), jnp.bfloat16),
            pltpu.VMEM((9 * Cout, th * Wp), jnp.bfloat16),
        ],
    )

    y = pl.pallas_call(
        _down_body,
        grid_spec=grid_spec,
        out_shape=jax.ShapeDtypeStruct((N, Cout, Hp * Wp), jnp.float32),
        compiler_params=pltpu.CompilerParams(
            dimension_semantics=("parallel", "parallel"),
            vmem_limit_bytes=56 * 2**20),
    )(x.reshape(N, Cin, Hp, 2 * W), w1m, b1b, w2m, b2b, sel)
    return y.reshape(N, Cout, Hp, Wp)


# speedup vs baseline: 1.7238x; 1.4823x over previous
import jax
import jax.numpy as jnp
from jax.experimental import pallas as pl
from jax.experimental.pallas import tpu as pltpu

_EPS = 1e-5


def _shift_w(v, d):
    if d == 0:
        return v
    z = jnp.zeros(v.shape[:-1] + (1,), v.dtype)
    if d == 1:
        return jnp.concatenate([v[..., 1:], z], axis=-1)
    return jnp.concatenate([z, v[..., :-1]], axis=-1)


def _down_body(x_hbm, w1_ref, b1_ref, w2_ref, b2_ref, sel_ref, o_ref,
               xin, a1, a2):
    n = pl.program_id(0)
    r = pl.program_id(1)
    num_r = pl.num_programs(1)
    cout = o_ref.shape[1]
    cin = xin.shape[0]
    wfull = xin.shape[2]
    wp = wfull // 2
    th = o_ref.shape[2] // wp

    @pl.when(r == 0)
    def _():
        xin[:, 0:8, :] = jnp.zeros((cin, 8, wfull), xin.dtype)
        pltpu.sync_copy(x_hbm.at[n, :, pl.ds(0, 2 * th + 8), :],
                        xin.at[:, pl.ds(8, 2 * th + 8), :])

    @pl.when(jnp.logical_and(r > 0, r < num_r - 1))
    def _():
        pltpu.sync_copy(x_hbm.at[n, :, pl.ds(2 * r * th - 8, 2 * th + 16), :],
                        xin.at[:, pl.ds(0, 2 * th + 16), :])

    @pl.when(jnp.logical_and(r == num_r - 1, num_r > 1))
    def _():
        pltpu.sync_copy(x_hbm.at[n, :, pl.ds(2 * r * th - 8, 2 * th + 8), :],
                        xin.at[:, pl.ds(0, 2 * th + 8), :])
        xin[:, 2 * th + 8:2 * th + 16, :] = jnp.zeros((cin, 8, wfull), xin.dtype)

    for i in range(th + 4):
        a = jnp.maximum(xin[:, 2 * i + 4, :], xin[:, 2 * i + 5, :])
        m = jnp.maximum(a, _shift_w(a, 1)).astype(jnp.bfloat16)
        p1 = jnp.dot(m, sel_ref[...],
                     preferred_element_type=jnp.float32).astype(jnp.bfloat16)
        pv = (_shift_w(p1, -1), p1, _shift_w(p1, 1))
        for ky in range(3):
            j = i - ky
            if 0 <= j < th + 2:
                for kx in range(3):
                    t = ky * 3 + kx
                    a1[t * cin:(t + 1) * cin, pl.ds(j * wp, wp)] = pv[kx]

    y1 = jnp.dot(w1_ref[...], a1[...], preferred_element_type=jnp.float32)
    y1 = jnp.maximum(y1 + b1_ref[:, 0:1], 0.0).astype(jnp.bfloat16)

    pos = jax.lax.broadcasted_iota(jnp.int32, (1, (th + 2) * wp), 1)
    zero = jnp.zeros((), jnp.bfloat16)
    posw = pos % wp
    y1_0 = jnp.where(posw == 0, zero, _shift_w(y1, -1))
    y1_2 = jnp.where(posw == wp - 1, zero, _shift_w(y1, 1))

    for kx, yv in ((0, y1_0), (1, y1), (2, y1_2)):
        for ky in range(3):
            t = ky * 3 + kx
            a2[t * cout:(t + 1) * cout, :] = yv[:, ky * wp:(ky + th) * wp]

    @pl.when(r == 0)
    def _():
        a2[0:3 * cout, 0:wp] = jnp.zeros((3 * cout, wp), a2.dtype)

    @pl.when(r == num_r - 1)
    def _():
        a2[6 * cout:9 * cout, (th - 1) * wp:th * wp] = jnp.zeros(
            (3 * cout, wp), a2.dtype)

    y2 = jnp.dot(w2_ref[...], a2[...], preferred_element_type=jnp.float32)
    y2 = jnp.maximum(y2 + b2_ref[:, 0:1], 0.0)
    o_ref[0] = y2.astype(jnp.bfloat16).astype(jnp.float32)


def _fold_bn(w, b, gamma, beta, mean, var, wp):
    kh, kw, cin, cout = w.shape
    scale = gamma / jnp.sqrt(var + _EPS)
    w_eff = w * scale[None, None, None, :]
    b_eff = (b - mean) * scale + beta
    wm = jnp.transpose(w_eff.reshape(kh * kw * cin, cout)).astype(jnp.bfloat16)
    bb = jnp.broadcast_to(b_eff[:, None], (cout, wp))
    return wm, bb


def kernel(x, w1, b1, gamma1, beta1, mean1, var1,
           w2, b2, gamma2, beta2, mean2, var2):
    N, Cin, H, W = x.shape
    Cout = w1.shape[-1]
    Hp, Wp = H // 2, W // 2

    th = min(32, Hp)
    while Hp % th:
        th -= 1
    R = Hp // th

    w1m, b1b = _fold_bn(w1, b1, gamma1, beta1, mean1, var1, Wp)
    w2m, b2b = _fold_bn(w2, b2, gamma2, beta2, mean2, var2, Wp)
    sel = (jnp.arange(W)[:, None] == 2 * jnp.arange(Wp)[None, :]
           ).astype(jnp.bfloat16)

    grid_spec = pltpu.PrefetchScalarGridSpec(
        num_scalar_prefetch=0,
        grid=(N, R),
        in_specs=[
            pl.BlockSpec(memory_space=pl.ANY),
            pl.BlockSpec((Cout, 9 * Cin), lambda n, r: (0, 0)),
            pl.BlockSpec((Cout, Wp), lambda n, r: (0, 0)),
            pl.BlockSpec((Cout, 9 * Cout), lambda n, r: (0, 0)),
            pl.BlockSpec((Cout, Wp), lambda n, r: (0, 0)),
            pl.BlockSpec((W, Wp), lambda n, r: (0, 0)),
        ],
        out_specs=pl.BlockSpec((1, Cout, th * Wp), lambda n, r: (n, 0, r)),
        scratch_shapes=[
            pltpu.VMEM((Cin, 2 * th + 16, W), jnp.float32),
            pltpu.VMEM((9 * Cin, (th + 2) * Wp), jnp.bfloat16),
            pltpu.VMEM((9 * Cout, th * Wp), jnp.bfloat16),
        ],
    )

    y = pl.pallas_call(
        _down_body,
        grid_spec=grid_spec,
        out_shape=jax.ShapeDtypeStruct((N, Cout, Hp * Wp), jnp.float32),
        compiler_params=pltpu.CompilerParams(
            dimension_semantics=("parallel", "parallel"),
            vmem_limit_bytes=56 * 2**20),
    )(x, w1m, b1b, w2m, b2b, sel)
    return y.reshape(N, Cout, Hp, Wp)
